# baseline (speedup 1.0000x reference)
import functools

import jax
import jax.numpy as jnp
from jax.experimental import pallas as pl
from jax.experimental.pallas import tpu as pltpu

_R = 2048
_C = 128
_L = _R * _C
_ROWS = 1152
_N_IMG = 16
_N_PIX = _ROWS * _C
_STAGES = 171


def _sortable(b):
    return b ^ ((b >> 31) & jnp.int32(0x7FFFFFFF))


import numpy as _np
_PAD_KEY = int(
    (int(_np.int32(_np.float32(-3e38).view(_np.int32))
         ^ ((_np.int32(_np.float32(-3e38).view(_np.int32)) >> 31) & _np.int32(0x7FFFFFFF)))
     & ~1)
)


def _lovasz_bce_kernel(pred_ref, tgt_ref, out_ref):
    img = pl.program_id(0)

    x = pred_ref[0]
    z = tgt_ref[0]

    bce_sum = jnp.sum(jnp.maximum(x, 0.0) - x * z
                      + jnp.log1p(jnp.exp(-jnp.abs(x))))
    p = jnp.sum(z)

    e = 1.0 - x * (2.0 * z - 1.0)
    kbits = _sortable(jax.lax.bitcast_convert_type(e, jnp.int32))
    key = (kbits & jnp.int32(~1)) | z.astype(jnp.int32)
    keys = jnp.concatenate(
        [key, jnp.full((_R - _ROWS, _C), _PAD_KEY, jnp.int32)], axis=0)

    i_full = (jax.lax.broadcasted_iota(jnp.int32, (_R, _C), 0)
              + _R * jax.lax.broadcasted_iota(jnp.int32, (_R, _C), 1))

    def stage_body(_, carry):
        v, j, k = carry
        jc = j >> 11

        def lane_partner(v, jc):
            up = pltpu.roll(v, _C - jc, axis=1)
            dn = pltpu.roll(v, jc, axis=1)
            return up, dn

        def sub_partner(v, j):
            up = pltpu.roll(v, _R - j, axis=0)
            dn = pltpu.roll(v, j, axis=0)
            return up, dn

        up, dn = jax.lax.cond(j >= _R,
                              lambda: lane_partner(v, jc),
                              lambda: sub_partner(v, j))
        lower = (i_full & j) == 0
        part = jnp.where(lower, up, dn)
        desc = (i_full & k) == 0
        take_max = jnp.logical_not(jnp.logical_xor(lower, desc))
        mn = jnp.minimum(v, part)
        mx = jnp.maximum(v, part)
        v = jnp.where(take_max, mx, mn)

        j2 = j >> 1
        nxt = j2 == 0
        k_new = jnp.where(nxt, k << 1, k)
        j_new = jnp.where(nxt, k, j2)
        return v, j_new, k_new

    v, _, _ = jax.lax.fori_loop(
        0, _STAGES, stage_body,
        (keys, jnp.int32(1), jnp.int32(2)))

    gt = (v & 1).astype(jnp.float32)
    e_sorted = jax.lax.bitcast_convert_type(_sortable(v), jnp.float32)

    cs = gt
    sh = 1
    while sh < _R:
        cs = cs + jnp.concatenate(
            [jnp.zeros((sh, _C), jnp.float32), cs[:-sh, :]], axis=0)
        sh *= 2
    tot = cs[_R - 1:_R, :]
    run = tot
    sh = 1
    while sh < _C:
        run = run + jnp.concatenate(
            [jnp.zeros((1, sh), jnp.float32), run[:, :-sh]], axis=1)
        sh *= 2
    excl = run - tot
    cs = cs + excl

    posn = (i_full + 1).astype(jnp.float32)
    jac = 1.0 - (p - cs) / (p + posn - cs)
    top = jnp.concatenate(
        [jnp.zeros((1, 1), jnp.float32), jac[_R - 1:_R, :-1]], axis=1)
    prevj = jnp.concatenate([top, jac[:-1, :]], axis=0)
    grad = jac - prevj
    lov = jnp.sum(jnp.maximum(e_sorted, 0.0) * grad)

    contrib = bce_sum / (_N_IMG * _N_PIX) + lov / _N_IMG

    @pl.when(img == 0)
    def _():
        out_ref[...] = jnp.zeros((1, 1), jnp.float32)

    out_ref[...] += jnp.full((1, 1), contrib, jnp.float32)


@jax.jit
def kernel(prediction, target):
    pred = prediction.reshape(_N_IMG, _ROWS, _C)
    tgt = target.reshape(_N_IMG, _ROWS, _C)
    out = pl.pallas_call(
        _lovasz_bce_kernel,
        grid=(_N_IMG,),
        in_specs=[
            pl.BlockSpec((1, _ROWS, _C), lambda i: (i, 0, 0)),
            pl.BlockSpec((1, _ROWS, _C), lambda i: (i, 0, 0)),
        ],
        out_specs=pl.BlockSpec((1, 1), lambda i: (0, 0)),
        out_shape=jax.ShapeDtypeStruct((1, 1), jnp.float32),
        compiler_params=pltpu.CompilerParams(
            dimension_semantics=("arbitrary",)),
    )(pred, tgt)
    return out[0, 0]


# speedup vs baseline: 8.1316x; 8.1316x over previous
import jax
import jax.numpy as jnp
import numpy as _np
from jax.experimental import pallas as pl
from jax.experimental.pallas import tpu as pltpu

_R = 2048
_C = 128
_L = _R * _C
_ROWS = 1152
_N_IMG = 16
_N_PIX = _ROWS * _C
_LOGL = 18


def _sortable(b):
    return b ^ ((b >> 31) & jnp.int32(0x7FFFFFFF))


_b = _np.float32(-3e38).view(_np.int32)
_PAD_KEY = int((_b ^ ((_b >> 31) & _np.int32(0x7FFFFFFF))) & ~_np.int32(1))


def _lovasz_bce_kernel(pred_ref, tgt_ref, out_ref, v_ref, desc_ref):
    img = pl.program_id(0)

    x = pred_ref[0]
    z = tgt_ref[0]

    bce_sum = jnp.sum(jnp.maximum(x, 0.0) - x * z
                      + jnp.log1p(jnp.exp(-jnp.abs(x))))
    p = jnp.sum(z)

    e = 1.0 - x * (2.0 * z - 1.0)
    kbits = _sortable(jax.lax.bitcast_convert_type(e, jnp.int32))
    v_ref[0:_ROWS, :] = (kbits & jnp.int32(~1)) | z.astype(jnp.int32)
    v_ref[_ROWS:_R, :] = jnp.full((_R - _ROWS, _C), _PAD_KEY, jnp.int32)

    r_iota = jax.lax.broadcasted_iota(jnp.int32, (_R, _C), 0)
    c_iota = jax.lax.broadcasted_iota(jnp.int32, (_R, _C), 1)
    i_full = r_iota + _R * c_iota

    def roll_stage(j):
        v = v_ref[...]
        desc = desc_ref[...] != 0
        if j >= _R:
            jc = j // _R
            up = pltpu.roll(v, _C - jc, axis=1)
            dn = pltpu.roll(v, jc, axis=1)
            lower = (c_iota & jc) == 0
        else:
            up = pltpu.roll(v, _R - j, axis=0)
            dn = pltpu.roll(v, j, axis=0)
            lower = (r_iota & j) == 0
        part = jnp.where(lower, up, dn)
        mn = jnp.minimum(v, part)
        mx = jnp.maximum(v, part)
        take_max = jnp.logical_not(jnp.logical_xor(lower, desc))
        v_ref[...] = jnp.where(take_max, mx, mn)

    def block_stage(j):
        for blk in range(_R // (2 * j)):
            base = blk * 2 * j
            a = v_ref[base:base + j, :]
            b = v_ref[base + j:base + 2 * j, :]
            dsc = desc_ref[base:base + j, :] != 0
            mn = jnp.minimum(a, b)
            mx = jnp.maximum(a, b)
            v_ref[base:base + j, :] = jnp.where(dsc, mx, mn)
            v_ref[base + j:base + 2 * j, :] = jnp.where(dsc, mn, mx)

    def level_body(m0, carry):
        k = jnp.int32(2) << m0
        desc_ref[...] = ((i_full & k) == 0).astype(jnp.int32)
        for j in [1 << t for t in range(_LOGL - 1, -1, -1)]:
            @pl.when(j < k)
            def _():
                if j >= _R or j < 8:
                    roll_stage(j)
                else:
                    block_stage(j)
        return carry

    jax.lax.fori_loop(0, _LOGL, level_body, 0)

    v = v_ref[...]
    gt = (v & 1).astype(jnp.float32)
    e_sorted = jax.lax.bitcast_convert_type(_sortable(v), jnp.float32)

    cs = gt
    sh = 1
    while sh < _R:
        cs = cs + jnp.concatenate(
            [jnp.zeros((sh, _C), jnp.float32), cs[:-sh, :]], axis=0)
        sh *= 2
    tot = cs[_R - 1:_R, :]
    run = tot
    sh = 1
    while sh < _C:
        run = run + jnp.concatenate(
            [jnp.zeros((1, sh), jnp.float32), run[:, :-sh]], axis=1)
        sh *= 2
    excl = run - tot
    cs = cs + excl

    posn = (i_full + 1).astype(jnp.float32)
    jac = 1.0 - (p - cs) / (p + posn - cs)
    top = jnp.concatenate(
        [jnp.zeros((1, 1), jnp.float32), jac[_R - 1:_R, :-1]], axis=1)
    prevj = jnp.concatenate([top, jac[:-1, :]], axis=0)
    grad = jac - prevj
    lov = jnp.sum(jnp.maximum(e_sorted, 0.0) * grad)

    contrib = bce_sum / (_N_IMG * _N_PIX) + lov / _N_IMG

    @pl.when(img == 0)
    def _():
        out_ref[...] = jnp.zeros((1, 1), jnp.float32)

    out_ref[...] += jnp.full((1, 1), contrib, jnp.float32)


@jax.jit
def kernel(prediction, target):
    pred = prediction.reshape(_N_IMG, _ROWS, _C)
    tgt = target.reshape(_N_IMG, _ROWS, _C)
    out = pl.pallas_call(
        _lovasz_bce_kernel,
        grid=(_N_IMG,),
        in_specs=[
            pl.BlockSpec((1, _ROWS, _C), lambda i: (i, 0, 0)),
            pl.BlockSpec((1, _ROWS, _C), lambda i: (i, 0, 0)),
        ],
        out_specs=pl.BlockSpec((1, 1), lambda i: (0, 0)),
        out_shape=jax.ShapeDtypeStruct((1, 1), jnp.float32),
        scratch_shapes=[
            pltpu.VMEM((_R, _C), jnp.int32),
            pltpu.VMEM((_R, _C), jnp.int32),
        ],
        compiler_params=pltpu.CompilerParams(
            dimension_semantics=("arbitrary",)),
    )(pred, tgt)
    return out[0, 0]


# speedup vs baseline: 9.5767x; 1.1777x over previous
import jax
import jax.numpy as jnp
import numpy as _np
from jax.experimental import pallas as pl
from jax.experimental.pallas import tpu as pltpu

_R = 2048
_C = 128
_RA = 1024
_ROWS = 1152
_N_IMG = 16
_N_PIX = _ROWS * _C


def _sortable(b):
    return b ^ ((b >> 31) & jnp.int32(0x7FFFFFFF))


_b = _np.float32(-3e38).view(_np.int32)
_PAD_KEY = int((_b ^ ((_b >> 31) & _np.int32(0x7FFFFFFF))) & ~_np.int32(1))


def _lovasz_bce_kernel(pred_ref, tgt_ref, out_ref, v_ref, desc_ref):
    img = pl.program_id(0)

    x = pred_ref[0]
    z = tgt_ref[0]

    bce_sum = jnp.sum(jnp.maximum(x, 0.0) - x * z
                      + jnp.log1p(jnp.exp(-jnp.abs(x))))
    p = jnp.sum(z)

    e = 1.0 - x * (2.0 * z - 1.0)
    kbits = _sortable(jax.lax.bitcast_convert_type(e, jnp.int32))
    key = (kbits & jnp.int32(~1)) | z.astype(jnp.int32)
    v_ref[0:_RA, :] = key[0:_RA, :]
    v_ref[_RA:_R, :] = jnp.full((_R - _RA, _C), _PAD_KEY, jnp.int32)

    rA = jax.lax.broadcasted_iota(jnp.int32, (_RA, _C), 0)
    cA = jax.lax.broadcasted_iota(jnp.int32, (_RA, _C), 1)
    iA = rA + _RA * cA

    def a_lane_stage(j):
        jc = j // _RA
        a = v_ref[0:_RA, :]
        desc = desc_ref[...] != 0
        up = pltpu.roll(a, _C - jc, axis=1)
        dn = pltpu.roll(a, jc, axis=1)
        lower = (cA & jc) == 0
        part = jnp.where(lower, up, dn)
        mn = jnp.minimum(a, part)
        mx = jnp.maximum(a, part)
        take_max = jnp.logical_not(jnp.logical_xor(lower, desc))
        v_ref[0:_RA, :] = jnp.where(take_max, mx, mn)

    def a_block_stage(j):
        for blk in range(_RA // (2 * j)):
            base = blk * 2 * j
            a = v_ref[base:base + j, :]
            b = v_ref[base + j:base + 2 * j, :]
            dsc = desc_ref[base:base + j, :] != 0
            mn = jnp.minimum(a, b)
            mx = jnp.maximum(a, b)
            v_ref[base:base + j, :] = jnp.where(dsc, mx, mn)
            v_ref[base + j:base + 2 * j, :] = jnp.where(dsc, mn, mx)

    def a_small_stages(strides):
        a = v_ref[0:_RA, :]
        desc = desc_ref[...] != 0
        for j in strides:
            up = pltpu.roll(a, _RA - j, axis=0)
            dn = pltpu.roll(a, j, axis=0)
            lower = (rA & j) == 0
            part = jnp.where(lower, up, dn)
            mn = jnp.minimum(a, part)
            mx = jnp.maximum(a, part)
            take_max = jnp.logical_not(jnp.logical_xor(lower, desc))
            a = jnp.where(take_max, mx, mn)
        v_ref[0:_RA, :] = a

    def a_level(m0, carry):
        k = jnp.int32(2) << m0
        desc_ref[...] = ((iA & k) == 0).astype(jnp.int32)
        for j in [1 << t for t in range(16, 9, -1)]:
            @pl.when(j < k)
            def _():
                a_lane_stage(j)
        for j in [512, 256, 128, 64, 32, 16, 8]:
            @pl.when(j < k)
            def _():
                a_block_stage(j)

        @pl.when(k > 4)
        def _():
            a_small_stages([4, 2, 1])

        @pl.when(k == 4)
        def _():
            a_small_stages([2, 1])

        @pl.when(k == 2)
        def _():
            a_small_stages([1])
        return carry

    jax.lax.fori_loop(0, 17, a_level, 0)

    rB = jax.lax.broadcasted_iota(jnp.int32, (_C, _C), 0)
    cB = jax.lax.broadcasted_iota(jnp.int32, (_C, _C), 1)
    iB = _C * rB + cB

    def b_level(m0, b):
        k = jnp.int32(2) << m0
        asc = (iB & k) == 0
        for j in [1 << t for t in range(13, -1, -1)]:
            if j >= _C:
                js = j // _C
                up = pltpu.roll(b, _C - js, axis=0)
                dn = pltpu.roll(b, js, axis=0)
                lower = (rB & js) == 0
            else:
                up = pltpu.roll(b, _C - j, axis=1)
                dn = pltpu.roll(b, j, axis=1)
                lower = (cB & j) == 0
            part = jnp.where(lower, up, dn)
            mn = jnp.minimum(b, part)
            mx = jnp.maximum(b, part)
            take_max = jnp.logical_xor(lower, asc)
            b = jnp.where((j < k), jnp.where(take_max, mx, mn), b)
        return b

    b_sorted = jax.lax.fori_loop(0, 14, b_level, key[_RA:_ROWS, :])

    bb = b_sorted.reshape(16, 8, _C)
    parts = [jnp.transpose(bb[:, a, :]) for a in range(8)]
    v_ref[_RA:_R, 112:128] = jnp.concatenate(parts, axis=0)

    lo = v_ref[0:_RA, :]
    hi = v_ref[_RA:_R, :]
    v_ref[0:_RA, :] = jnp.maximum(lo, hi)
    v_ref[_RA:_R, :] = jnp.minimum(lo, hi)

    c_full = jax.lax.broadcasted_iota(jnp.int32, (_R, _C), 1)
    r_full = jax.lax.broadcasted_iota(jnp.int32, (_R, _C), 0)
    v = v_ref[...]
    for jc in [64, 32, 16, 8, 4, 2, 1]:
        up = pltpu.roll(v, _C - jc, axis=1)
        dn = pltpu.roll(v, jc, axis=1)
        lower = (c_full & jc) == 0
        part = jnp.where(lower, up, dn)
        v = jnp.where(lower, jnp.maximum(v, part), jnp.minimum(v, part))
    v_ref[...] = v

    for j in [512, 256, 128, 64, 32, 16, 8]:
        for blk in range(_R // (2 * j)):
            base = blk * 2 * j
            a = v_ref[base:base + j, :]
            b = v_ref[base + j:base + 2 * j, :]
            v_ref[base:base + j, :] = jnp.maximum(a, b)
            v_ref[base + j:base + 2 * j, :] = jnp.minimum(a, b)

    v = v_ref[...]
    for j in [4, 2, 1]:
        up = pltpu.roll(v, _R - j, axis=0)
        dn = pltpu.roll(v, j, axis=0)
        lower = (r_full & j) == 0
        part = jnp.where(lower, up, dn)
        v = jnp.where(lower, jnp.maximum(v, part), jnp.minimum(v, part))

    gt = (v & 1).astype(jnp.float32)
    e_sorted = jax.lax.bitcast_convert_type(_sortable(v), jnp.float32)

    cs = gt
    sh = 1
    while sh < _RA:
        blk = jnp.concatenate(
            [jnp.zeros((sh, _C), jnp.float32), cs[:_RA - sh, :],
             jnp.zeros((sh, _C), jnp.float32), cs[_RA:_R - sh, :]], axis=0)
        cs = cs + blk
        sh *= 2
    tot_lo = cs[_RA - 1:_RA, :]
    tot_hi = cs[_R - 1:_R, :]
    run_lo = tot_lo
    run_hi = tot_hi
    sh = 1
    while sh < _C:
        run_lo = run_lo + jnp.concatenate(
            [jnp.zeros((1, sh), jnp.float32), run_lo[:, :-sh]], axis=1)
        run_hi = run_hi + jnp.concatenate(
            [jnp.zeros((1, sh), jnp.float32), run_hi[:, :-sh]], axis=1)
        sh *= 2
    all_lo = run_lo[0:1, _C - 1:_C]
    excl_lo = run_lo - tot_lo
    excl_hi = run_hi - tot_hi + all_lo
    cs = cs + jnp.concatenate(
        [jnp.broadcast_to(excl_lo, (_RA, _C)),
         jnp.broadcast_to(excl_hi, (_RA, _C))], axis=0)

    i_half = (rA_mod := (r_full & (_RA - 1))) + _RA * c_full
    i_glob = i_half + jnp.where(r_full >= _RA, _RA * _C, 0)
    posn = (i_glob + 1).astype(jnp.float32)
    jac = 1.0 - (p - cs) / (p + posn - cs)
    top_lo = jnp.concatenate(
        [jnp.zeros((1, 1), jnp.float32), jac[_RA - 1:_RA, :-1]], axis=1)
    top_hi = jnp.concatenate(
        [jac[_RA - 1:_RA, _C - 1:_C], jac[_R - 1:_R, :-1]], axis=1)
    prevj = jnp.concatenate(
        [top_lo, jac[0:_RA - 1, :], top_hi, jac[_RA:_R - 1, :]], axis=0)
    grad = jac - prevj
    lov = jnp.sum(jnp.maximum(e_sorted, 0.0) * grad)

    contrib = bce_sum / (_N_IMG * _N_PIX) + lov / _N_IMG

    @pl.when(img == 0)
    def _():
        out_ref[...] = jnp.zeros((1, 1), jnp.float32)

    out_ref[...] += jnp.full((1, 1), contrib, jnp.float32)


@jax.jit
def kernel(prediction, target):
    pred = prediction.reshape(_N_IMG, _ROWS, _C)
    tgt = target.reshape(_N_IMG, _ROWS, _C)
    out = pl.pallas_call(
        _lovasz_bce_kernel,
        grid=(_N_IMG,),
        in_specs=[
            pl.BlockSpec((1, _ROWS, _C), lambda i: (i, 0, 0)),
            pl.BlockSpec((1, _ROWS, _C), lambda i: (i, 0, 0)),
        ],
        out_specs=pl.BlockSpec((1, 1), lambda i: (0, 0)),
        out_shape=jax.ShapeDtypeStruct((1, 1), jnp.float32),
        scratch_shapes=[
            pltpu.VMEM((_R, _C), jnp.int32),
            pltpu.VMEM((_RA, _C), jnp.int32),
        ],
        compiler_params=pltpu.CompilerParams(
            dimension_semantics=("arbitrary",)),
    )(pred, tgt)
    return out[0, 0]


# speedup vs baseline: 10.6496x; 1.1120x over previous
import jax
import jax.numpy as jnp
import numpy as _np
from jax.experimental import pallas as pl
from jax.experimental.pallas import tpu as pltpu

_R = 2048
_C = 128
_RA = 1024
_ROWS = 1152
_N_IMG = 16
_N_PIX = _ROWS * _C


def _sortable(b):
    return b ^ ((b >> 31) & jnp.int32(0x7FFFFFFF))


_b = _np.float32(-3e38).view(_np.int32)
_PAD_KEY = int((_b ^ ((_b >> 31) & _np.int32(0x7FFFFFFF))) & ~_np.int32(1))


def _lovasz_bce_kernel(pred_ref, tgt_ref, out_ref, v_ref):
    img = pl.program_id(0)

    x = pred_ref[0]
    z = tgt_ref[0]

    bce_sum = jnp.sum(jnp.maximum(x, 0.0) - x * z
                      + jnp.log1p(jnp.exp(-jnp.abs(x))))
    p = jnp.sum(z)

    e = 1.0 - x * (2.0 * z - 1.0)
    kbits = _sortable(jax.lax.bitcast_convert_type(e, jnp.int32))
    key = (kbits & jnp.int32(~1)) | z.astype(jnp.int32)
    v_ref[0:_RA, :] = key[0:_RA, :]
    v_ref[_RA:_R, :] = jnp.full((_R - _RA, _C), _PAD_KEY, jnp.int32)

    rA = jax.lax.broadcasted_iota(jnp.int32, (_RA, _C), 0)
    cA = jax.lax.broadcasted_iota(jnp.int32, (_RA, _C), 1)
    iA = rA + _RA * cA

    def a_small_stages(strides, desc):
        a = v_ref[0:_RA, :]
        for j in strides:
            up = pltpu.roll(a, _RA - j, axis=0)
            dn = pltpu.roll(a, j, axis=0)
            lower = (rA & j) == 0
            part = jnp.where(lower, up, dn)
            mn = jnp.minimum(a, part)
            mx = jnp.maximum(a, part)
            take_max = jnp.logical_not(jnp.logical_xor(lower, desc))
            a = jnp.where(take_max, mx, mn)
        v_ref[0:_RA, :] = a

    def a_level_lo(m0, carry):
        k = jnp.int32(2) << m0
        for j in [256, 128, 64, 32, 16, 8]:
            @pl.when(j < k)
            def _():
                for blk in range(_RA // (2 * j)):
                    base = blk * 2 * j
                    dsc = (base & k) == 0
                    a = v_ref[base:base + j, :]
                    b = v_ref[base + j:base + 2 * j, :]
                    mn = jnp.minimum(a, b)
                    mx = jnp.maximum(a, b)
                    v_ref[base:base + j, :] = jnp.where(dsc, mx, mn)
                    v_ref[base + j:base + 2 * j, :] = jnp.where(dsc, mn, mx)

        @pl.when(k > 4)
        def _():
            a_small_stages([4, 2, 1], (rA & k) == 0)

        @pl.when(k == 4)
        def _():
            a_small_stages([2, 1], (rA & k) == 0)

        @pl.when(k == 2)
        def _():
            a_small_stages([1], (rA & k) == 0)
        return carry

    jax.lax.fori_loop(0, 9, a_level_lo, 0)

    def a_level_hi(m0, carry):
        k = jnp.int32(2) << (m0 + 9)
        kc = k >> 10
        for j in [1 << t for t in range(16, 9, -1)]:
            @pl.when(j < k)
            def _():
                jc = j // _RA
                a = v_ref[0:_RA, :]
                up = pltpu.roll(a, _C - jc, axis=1)
                dn = pltpu.roll(a, jc, axis=1)
                lower = (cA & jc) == 0
                desc = (cA & kc) == 0
                part = jnp.where(lower, up, dn)
                mn = jnp.minimum(a, part)
                mx = jnp.maximum(a, part)
                take_max = jnp.logical_not(jnp.logical_xor(lower, desc))
                v_ref[0:_RA, :] = jnp.where(take_max, mx, mn)
        for j in [512, 256, 128, 64, 32, 16, 8]:
            for blk in range(_RA // (2 * j)):
                base = blk * 2 * j
                a = v_ref[base:base + j, :]
                b = v_ref[base + j:base + 2 * j, :]
                dsc = (jax.lax.broadcasted_iota(jnp.int32, (j, _C), 1)
                       & kc) == 0
                mn = jnp.minimum(a, b)
                mx = jnp.maximum(a, b)
                v_ref[base:base + j, :] = jnp.where(dsc, mx, mn)
                v_ref[base + j:base + 2 * j, :] = jnp.where(dsc, mn, mx)

        a_small_stages([4, 2, 1], (cA & kc) == 0)
        return carry

    jax.lax.fori_loop(0, 8, a_level_hi, 0)

    rB = jax.lax.broadcasted_iota(jnp.int32, (_C, _C), 0)
    cB = jax.lax.broadcasted_iota(jnp.int32, (_C, _C), 1)

    b = key[_RA:_ROWS, :]
    for m in range(1, 15):
        kB = 1 << m
        for t in range(m - 1, -1, -1):
            j = 1 << t
            if j >= _C:
                js = j // _C
                up = pltpu.roll(b, _C - js, axis=0)
                dn = pltpu.roll(b, js, axis=0)
                lower = (rB & js) == 0
                asc = (rB & (kB // _C)) == 0 if kB >= _C else (cB & kB) == 0
            else:
                up = pltpu.roll(b, _C - j, axis=1)
                dn = pltpu.roll(b, j, axis=1)
                lower = (cB & j) == 0
                asc = (rB & (kB // _C)) == 0 if kB >= _C else (cB & kB) == 0
            part = jnp.where(lower, up, dn)
            mn = jnp.minimum(b, part)
            mx = jnp.maximum(b, part)
            take_max = jnp.logical_xor(lower, asc)
            b = jnp.where(take_max, mx, mn)
    b_sorted = b

    bb = b_sorted.reshape(16, 8, _C)
    parts = [jnp.transpose(bb[:, a, :]) for a in range(8)]
    v_ref[_RA:_R, 112:128] = jnp.concatenate(parts, axis=0)

    lo = v_ref[0:_RA, :]
    hi = v_ref[_RA:_R, :]
    v_ref[0:_RA, :] = jnp.maximum(lo, hi)
    v_ref[_RA:_R, :] = jnp.minimum(lo, hi)

    c_full = jax.lax.broadcasted_iota(jnp.int32, (_R, _C), 1)
    r_full = jax.lax.broadcasted_iota(jnp.int32, (_R, _C), 0)
    v = v_ref[...]
    for jc in [64, 32, 16, 8, 4, 2, 1]:
        up = pltpu.roll(v, _C - jc, axis=1)
        dn = pltpu.roll(v, jc, axis=1)
        lower = (c_full & jc) == 0
        part = jnp.where(lower, up, dn)
        v = jnp.where(lower, jnp.maximum(v, part), jnp.minimum(v, part))
    v_ref[...] = v

    for j in [512, 256, 128, 64, 32, 16, 8]:
        for blk in range(_R // (2 * j)):
            base = blk * 2 * j
            a = v_ref[base:base + j, :]
            b = v_ref[base + j:base + 2 * j, :]
            v_ref[base:base + j, :] = jnp.maximum(a, b)
            v_ref[base + j:base + 2 * j, :] = jnp.minimum(a, b)

    v = v_ref[...]
    for j in [4, 2, 1]:
        up = pltpu.roll(v, _R - j, axis=0)
        dn = pltpu.roll(v, j, axis=0)
        lower = (r_full & j) == 0
        part = jnp.where(lower, up, dn)
        v = jnp.where(lower, jnp.maximum(v, part), jnp.minimum(v, part))

    gt = (v & 1).astype(jnp.float32)
    e_sorted = jax.lax.bitcast_convert_type(_sortable(v), jnp.float32)

    cs = gt
    sh = 1
    while sh < _RA:
        blk = jnp.concatenate(
            [jnp.zeros((sh, _C), jnp.float32), cs[:_RA - sh, :],
             jnp.zeros((sh, _C), jnp.float32), cs[_RA:_R - sh, :]], axis=0)
        cs = cs + blk
        sh *= 2
    tot_lo = cs[_RA - 1:_RA, :]
    tot_hi = cs[_R - 1:_R, :]
    run_lo = tot_lo
    run_hi = tot_hi
    sh = 1
    while sh < _C:
        run_lo = run_lo + jnp.concatenate(
            [jnp.zeros((1, sh), jnp.float32), run_lo[:, :-sh]], axis=1)
        run_hi = run_hi + jnp.concatenate(
            [jnp.zeros((1, sh), jnp.float32), run_hi[:, :-sh]], axis=1)
        sh *= 2
    all_lo = run_lo[0:1, _C - 1:_C]
    excl_lo = run_lo - tot_lo
    excl_hi = run_hi - tot_hi + all_lo
    cs = cs + jnp.concatenate(
        [jnp.broadcast_to(excl_lo, (_RA, _C)),
         jnp.broadcast_to(excl_hi, (_RA, _C))], axis=0)

    i_half = (rA_mod := (r_full & (_RA - 1))) + _RA * c_full
    i_glob = i_half + jnp.where(r_full >= _RA, _RA * _C, 0)
    posn = (i_glob + 1).astype(jnp.float32)
    jac = 1.0 - (p - cs) / (p + posn - cs)
    top_lo = jnp.concatenate(
        [jnp.zeros((1, 1), jnp.float32), jac[_RA - 1:_RA, :-1]], axis=1)
    top_hi = jnp.concatenate(
        [jac[_RA - 1:_RA, _C - 1:_C], jac[_R - 1:_R, :-1]], axis=1)
    prevj = jnp.concatenate(
        [top_lo, jac[0:_RA - 1, :], top_hi, jac[_RA:_R - 1, :]], axis=0)
    grad = jac - prevj
    lov = jnp.sum(jnp.maximum(e_sorted, 0.0) * grad)

    contrib = bce_sum / (_N_IMG * _N_PIX) + lov / _N_IMG

    @pl.when(img == 0)
    def _():
        out_ref[...] = jnp.zeros((1, 1), jnp.float32)

    out_ref[...] += jnp.full((1, 1), contrib, jnp.float32)


@jax.jit
def kernel(prediction, target):
    pred = prediction.reshape(_N_IMG, _ROWS, _C)
    tgt = target.reshape(_N_IMG, _ROWS, _C)
    out = pl.pallas_call(
        _lovasz_bce_kernel,
        grid=(_N_IMG,),
        in_specs=[
            pl.BlockSpec((1, _ROWS, _C), lambda i: (i, 0, 0)),
            pl.BlockSpec((1, _ROWS, _C), lambda i: (i, 0, 0)),
        ],
        out_specs=pl.BlockSpec((1, 1), lambda i: (0, 0)),
        out_shape=jax.ShapeDtypeStruct((1, 1), jnp.float32),
        scratch_shapes=[
            pltpu.VMEM((_R, _C), jnp.int32),
        ],
        compiler_params=pltpu.CompilerParams(
            dimension_semantics=("arbitrary",)),
    )(pred, tgt)
    return out[0, 0]


# speedup vs baseline: 10.7196x; 1.0066x over previous
import jax
import jax.numpy as jnp
import numpy as _np
from jax.experimental import pallas as pl
from jax.experimental.pallas import tpu as pltpu

_R = 2048
_C = 128
_RA = 1024
_ROWS = 1152
_N_IMG = 16
_N_PIX = _ROWS * _C


def _sortable(b):
    return b ^ ((b >> 31) & jnp.int32(0x7FFFFFFF))


_b = _np.float32(-3e38).view(_np.int32)
_PAD_KEY = int((_b ^ ((_b >> 31) & _np.int32(0x7FFFFFFF))) & ~_np.int32(1))


def _lovasz_bce_kernel(pred_ref, tgt_ref, out_ref, v_ref):
    img = pl.program_id(0)

    x = pred_ref[0]
    z = tgt_ref[0]

    bce_sum = jnp.sum(jnp.maximum(x, 0.0) - x * z
                      + jnp.log1p(jnp.exp(-jnp.abs(x))))
    p = jnp.sum(z)

    e = 1.0 - x * (2.0 * z - 1.0)
    kbits = _sortable(jax.lax.bitcast_convert_type(e, jnp.int32))
    key = (kbits & jnp.int32(~1)) | z.astype(jnp.int32)
    v_ref[0:_RA, :] = key[0:_RA, :]
    v_ref[_RA:_R, :] = jnp.full((_R - _RA, _C), _PAD_KEY, jnp.int32)

    rA = jax.lax.broadcasted_iota(jnp.int32, (_RA, _C), 0)
    cA = jax.lax.broadcasted_iota(jnp.int32, (_RA, _C), 1)
    iA = rA + _RA * cA

    def a_small_stages(strides, desc):
        a = v_ref[0:_RA, :]
        for j in strides:
            up = pltpu.roll(a, _RA - j, axis=0)
            dn = pltpu.roll(a, j, axis=0)
            lower = (rA & j) == 0
            part = jnp.where(lower, up, dn)
            mn = jnp.minimum(a, part)
            mx = jnp.maximum(a, part)
            take_max = jnp.logical_not(jnp.logical_xor(lower, desc))
            a = jnp.where(take_max, mx, mn)
        v_ref[0:_RA, :] = a

    def a_level_lo(m0, carry):
        k = jnp.int32(2) << m0
        for j in [256, 128, 64, 32, 16, 8]:
            @pl.when(j < k)
            def _():
                for blk in range(_RA // (2 * j)):
                    base = blk * 2 * j
                    dsc = (base & k) == 0
                    a = v_ref[base:base + j, :]
                    b = v_ref[base + j:base + 2 * j, :]
                    mn = jnp.minimum(a, b)
                    mx = jnp.maximum(a, b)
                    v_ref[base:base + j, :] = jnp.where(dsc, mx, mn)
                    v_ref[base + j:base + 2 * j, :] = jnp.where(dsc, mn, mx)

        @pl.when(k > 4)
        def _():
            a_small_stages([4, 2, 1], (rA & k) == 0)

        @pl.when(k == 4)
        def _():
            a_small_stages([2, 1], (rA & k) == 0)

        @pl.when(k == 2)
        def _():
            a_small_stages([1], (rA & k) == 0)
        return carry

    jax.lax.fori_loop(0, 9, a_level_lo, 0)

    def a_level_hi(m0, carry):
        k = jnp.int32(2) << (m0 + 9)
        kc = k >> 10
        for j in [1 << t for t in range(16, 9, -1)]:
            @pl.when(j < k)
            def _():
                jc = j // _RA
                a = v_ref[0:_RA, :]
                up = pltpu.roll(a, _C - jc, axis=1)
                dn = pltpu.roll(a, jc, axis=1)
                lower = (cA & jc) == 0
                desc = (cA & kc) == 0
                part = jnp.where(lower, up, dn)
                mn = jnp.minimum(a, part)
                mx = jnp.maximum(a, part)
                take_max = jnp.logical_not(jnp.logical_xor(lower, desc))
                v_ref[0:_RA, :] = jnp.where(take_max, mx, mn)
        for j in [512, 128, 32]:
            j2 = j // 2
            dsc = (jax.lax.broadcasted_iota(jnp.int32, (j2, _C), 1)
                   & kc) == 0
            for blk in range(_RA // (2 * j)):
                base = blk * 2 * j
                a1 = v_ref[base:base + j2, :]
                a2 = v_ref[base + j2:base + j, :]
                b1 = v_ref[base + j:base + j + j2, :]
                b2 = v_ref[base + j + j2:base + 2 * j, :]
                na1 = jnp.where(dsc, jnp.maximum(a1, b1), jnp.minimum(a1, b1))
                nb1 = jnp.where(dsc, jnp.minimum(a1, b1), jnp.maximum(a1, b1))
                na2 = jnp.where(dsc, jnp.maximum(a2, b2), jnp.minimum(a2, b2))
                nb2 = jnp.where(dsc, jnp.minimum(a2, b2), jnp.maximum(a2, b2))
                v_ref[base:base + j2, :] = jnp.where(
                    dsc, jnp.maximum(na1, na2), jnp.minimum(na1, na2))
                v_ref[base + j2:base + j, :] = jnp.where(
                    dsc, jnp.minimum(na1, na2), jnp.maximum(na1, na2))
                v_ref[base + j:base + j + j2, :] = jnp.where(
                    dsc, jnp.maximum(nb1, nb2), jnp.minimum(nb1, nb2))
                v_ref[base + j + j2:base + 2 * j, :] = jnp.where(
                    dsc, jnp.minimum(nb1, nb2), jnp.maximum(nb1, nb2))
        dsc8 = (jax.lax.broadcasted_iota(jnp.int32, (8, _C), 1) & kc) == 0
        for blk in range(_RA // 16):
            base = blk * 16
            a = v_ref[base:base + 8, :]
            b = v_ref[base + 8:base + 16, :]
            mn = jnp.minimum(a, b)
            mx = jnp.maximum(a, b)
            v_ref[base:base + 8, :] = jnp.where(dsc8, mx, mn)
            v_ref[base + 8:base + 16, :] = jnp.where(dsc8, mn, mx)

        a_small_stages([4, 2, 1], (cA & kc) == 0)
        return carry

    jax.lax.fori_loop(0, 8, a_level_hi, 0)

    rB = jax.lax.broadcasted_iota(jnp.int32, (_C, _C), 0)
    cB = jax.lax.broadcasted_iota(jnp.int32, (_C, _C), 1)

    b = key[_RA:_ROWS, :]
    for m in range(1, 15):
        kB = 1 << m
        for t in range(m - 1, -1, -1):
            j = 1 << t
            if j >= _C:
                js = j // _C
                up = pltpu.roll(b, _C - js, axis=0)
                dn = pltpu.roll(b, js, axis=0)
                lower = (rB & js) == 0
                asc = (rB & (kB // _C)) == 0 if kB >= _C else (cB & kB) == 0
            else:
                up = pltpu.roll(b, _C - j, axis=1)
                dn = pltpu.roll(b, j, axis=1)
                lower = (cB & j) == 0
                asc = (rB & (kB // _C)) == 0 if kB >= _C else (cB & kB) == 0
            part = jnp.where(lower, up, dn)
            mn = jnp.minimum(b, part)
            mx = jnp.maximum(b, part)
            take_max = jnp.logical_xor(lower, asc)
            b = jnp.where(take_max, mx, mn)
    b_sorted = b

    bb = b_sorted.reshape(16, 8, _C)
    parts = [jnp.transpose(bb[:, a, :]) for a in range(8)]
    v_ref[_RA:_R, 112:128] = jnp.concatenate(parts, axis=0)

    c_full = jax.lax.broadcasted_iota(jnp.int32, (_R, _C), 1)
    r_full = jax.lax.broadcasted_iota(jnp.int32, (_R, _C), 0)

    v = v_ref[...]
    lo = v[0:_RA, :]
    hi = v[_RA:_R, :]
    v = jnp.concatenate([jnp.maximum(lo, hi), jnp.minimum(lo, hi)], axis=0)
    for jc in [64, 32, 16, 8, 4, 2, 1]:
        up = pltpu.roll(v, _C - jc, axis=1)
        dn = pltpu.roll(v, jc, axis=1)
        lower = (c_full & jc) == 0
        part = jnp.where(lower, up, dn)
        v = jnp.where(lower, jnp.maximum(v, part), jnp.minimum(v, part))
    v_ref[...] = v

    for j in [512, 128, 32]:
        j2 = j // 2
        for blk in range(_R // (2 * j)):
            base = blk * 2 * j
            a1 = v_ref[base:base + j2, :]
            a2 = v_ref[base + j2:base + j, :]
            b1 = v_ref[base + j:base + j + j2, :]
            b2 = v_ref[base + j + j2:base + 2 * j, :]
            na1 = jnp.maximum(a1, b1)
            nb1 = jnp.minimum(a1, b1)
            na2 = jnp.maximum(a2, b2)
            nb2 = jnp.minimum(a2, b2)
            v_ref[base:base + j2, :] = jnp.maximum(na1, na2)
            v_ref[base + j2:base + j, :] = jnp.minimum(na1, na2)
            v_ref[base + j:base + j + j2, :] = jnp.maximum(nb1, nb2)
            v_ref[base + j + j2:base + 2 * j, :] = jnp.minimum(nb1, nb2)
    for blk in range(_R // 16):
        base = blk * 16
        a = v_ref[base:base + 8, :]
        b = v_ref[base + 8:base + 16, :]
        v_ref[base:base + 8, :] = jnp.maximum(a, b)
        v_ref[base + 8:base + 16, :] = jnp.minimum(a, b)

    v = v_ref[...]
    for j in [4, 2, 1]:
        up = pltpu.roll(v, _R - j, axis=0)
        dn = pltpu.roll(v, j, axis=0)
        lower = (r_full & j) == 0
        part = jnp.where(lower, up, dn)
        v = jnp.where(lower, jnp.maximum(v, part), jnp.minimum(v, part))

    gt = (v & 1).astype(jnp.float32)
    e_sorted = jax.lax.bitcast_convert_type(_sortable(v), jnp.float32)

    cs = gt
    sh = 1
    while sh < _RA:
        blk = jnp.concatenate(
            [jnp.zeros((sh, _C), jnp.float32), cs[:_RA - sh, :],
             jnp.zeros((sh, _C), jnp.float32), cs[_RA:_R - sh, :]], axis=0)
        cs = cs + blk
        sh *= 2
    tot_lo = cs[_RA - 1:_RA, :]
    tot_hi = cs[_R - 1:_R, :]
    run_lo = tot_lo
    run_hi = tot_hi
    sh = 1
    while sh < _C:
        run_lo = run_lo + jnp.concatenate(
            [jnp.zeros((1, sh), jnp.float32), run_lo[:, :-sh]], axis=1)
        run_hi = run_hi + jnp.concatenate(
            [jnp.zeros((1, sh), jnp.float32), run_hi[:, :-sh]], axis=1)
        sh *= 2
    all_lo = run_lo[0:1, _C - 1:_C]
    excl_lo = run_lo - tot_lo
    excl_hi = run_hi - tot_hi + all_lo
    cs = cs + jnp.concatenate(
        [jnp.broadcast_to(excl_lo, (_RA, _C)),
         jnp.broadcast_to(excl_hi, (_RA, _C))], axis=0)

    i_half = (rA_mod := (r_full & (_RA - 1))) + _RA * c_full
    i_glob = i_half + jnp.where(r_full >= _RA, _RA * _C, 0)
    posn = (i_glob + 1).astype(jnp.float32)
    jac = 1.0 - (p - cs) / (p + posn - cs)
    top_lo = jnp.concatenate(
        [jnp.zeros((1, 1), jnp.float32), jac[_RA - 1:_RA, :-1]], axis=1)
    top_hi = jnp.concatenate(
        [jac[_RA - 1:_RA, _C - 1:_C], jac[_R - 1:_R, :-1]], axis=1)
    prevj = jnp.concatenate(
        [top_lo, jac[0:_RA - 1, :], top_hi, jac[_RA:_R - 1, :]], axis=0)
    grad = jac - prevj
    lov = jnp.sum(jnp.maximum(e_sorted, 0.0) * grad)

    contrib = bce_sum / (_N_IMG * _N_PIX) + lov / _N_IMG

    @pl.when(img == 0)
    def _():
        out_ref[...] = jnp.zeros((1, 1), jnp.float32)

    out_ref[...] += jnp.full((1, 1), contrib, jnp.float32)


@jax.jit
def kernel(prediction, target):
    pred = prediction.reshape(_N_IMG, _ROWS, _C)
    tgt = target.reshape(_N_IMG, _ROWS, _C)
    out = pl.pallas_call(
        _lovasz_bce_kernel,
        grid=(_N_IMG,),
        in_specs=[
            pl.BlockSpec((1, _ROWS, _C), lambda i: (i, 0, 0)),
            pl.BlockSpec((1, _ROWS, _C), lambda i: (i, 0, 0)),
        ],
        out_specs=pl.BlockSpec((1, 1), lambda i: (0, 0)),
        out_shape=jax.ShapeDtypeStruct((1, 1), jnp.float32),
        scratch_shapes=[
            pltpu.VMEM((_R, _C), jnp.int32),
        ],
        compiler_params=pltpu.CompilerParams(
            dimension_semantics=("arbitrary",)),
    )(pred, tgt)
    return out[0, 0]


# speedup vs baseline: 10.7594x; 1.0037x over previous
import jax
import jax.numpy as jnp
import numpy as _np
from jax.experimental import pallas as pl
from jax.experimental.pallas import tpu as pltpu

_R = 2048
_C = 128
_RA = 1024
_ROWS = 1152
_N_IMG = 16
_N_PIX = _ROWS * _C


def _sortable(b):
    return b ^ ((b >> 31) & jnp.int32(0x7FFFFFFF))


_b = _np.float32(-3e38).view(_np.int32)
_PAD_KEY = int((_b ^ ((_b >> 31) & _np.int32(0x7FFFFFFF))) & ~_np.int32(1))


def _lovasz_bce_kernel(pred_ref, tgt_ref, out_ref, v_ref):
    img = pl.program_id(0)

    x = pred_ref[0]
    z = tgt_ref[0]

    bce_sum = jnp.sum(jnp.maximum(x, 0.0) - x * z
                      + jnp.log1p(jnp.exp(-jnp.abs(x))))
    p = jnp.sum(z)

    e = 1.0 - x * (2.0 * z - 1.0)
    kbits = _sortable(jax.lax.bitcast_convert_type(e, jnp.int32))
    key = (kbits & jnp.int32(~1)) | z.astype(jnp.int32)
    v_ref[0:_RA, :] = key[0:_RA, :]
    v_ref[_RA:_R, :] = jnp.full((_R - _RA, _C), _PAD_KEY, jnp.int32)

    rA = jax.lax.broadcasted_iota(jnp.int32, (_RA, _C), 0)
    cA = jax.lax.broadcasted_iota(jnp.int32, (_RA, _C), 1)
    iA = rA + _RA * cA

    def a_small_stages(strides, desc):
        a = v_ref[0:_RA, :]
        for j in strides:
            up = pltpu.roll(a, _RA - j, axis=0)
            dn = pltpu.roll(a, j, axis=0)
            lower = (rA & j) == 0
            part = jnp.where(lower, up, dn)
            mn = jnp.minimum(a, part)
            mx = jnp.maximum(a, part)
            take_max = jnp.logical_not(jnp.logical_xor(lower, desc))
            a = jnp.where(take_max, mx, mn)
        v_ref[0:_RA, :] = a

    def a_level_lo(m0, carry):
        k = jnp.int32(2) << m0
        for j in [256, 128, 64, 32, 16, 8]:
            @pl.when(j < k)
            def _():
                for blk in range(_RA // (2 * j)):
                    base = blk * 2 * j
                    dsc = (base & k) == 0
                    a = v_ref[base:base + j, :]
                    b = v_ref[base + j:base + 2 * j, :]
                    mn = jnp.minimum(a, b)
                    mx = jnp.maximum(a, b)
                    v_ref[base:base + j, :] = jnp.where(dsc, mx, mn)
                    v_ref[base + j:base + 2 * j, :] = jnp.where(dsc, mn, mx)

        @pl.when(k > 4)
        def _():
            a_small_stages([4, 2, 1], (rA & k) == 0)

        @pl.when(k == 4)
        def _():
            a_small_stages([2, 1], (rA & k) == 0)

        @pl.when(k == 2)
        def _():
            a_small_stages([1], (rA & k) == 0)
        return carry

    jax.lax.fori_loop(0, 9, a_level_lo, 0)

    def a_level_hi(m0, carry):
        k = jnp.int32(2) << (m0 + 9)
        kc = k >> 10
        for j in [1 << t for t in range(16, 9, -1)]:
            @pl.when(j < k)
            def _():
                jc = j // _RA
                a = v_ref[0:_RA, :]
                up = pltpu.roll(a, _C - jc, axis=1)
                dn = pltpu.roll(a, jc, axis=1)
                lower = (cA & jc) == 0
                desc = (cA & kc) == 0
                part = jnp.where(lower, up, dn)
                mn = jnp.minimum(a, part)
                mx = jnp.maximum(a, part)
                take_max = jnp.logical_not(jnp.logical_xor(lower, desc))
                v_ref[0:_RA, :] = jnp.where(take_max, mx, mn)
        for j in [512, 128, 32]:
            j2 = j // 2
            dsc = (jax.lax.broadcasted_iota(jnp.int32, (j2, _C), 1)
                   & kc) == 0
            for blk in range(_RA // (2 * j)):
                base = blk * 2 * j
                a1 = v_ref[base:base + j2, :]
                a2 = v_ref[base + j2:base + j, :]
                b1 = v_ref[base + j:base + j + j2, :]
                b2 = v_ref[base + j + j2:base + 2 * j, :]
                na1 = jnp.where(dsc, jnp.maximum(a1, b1), jnp.minimum(a1, b1))
                nb1 = jnp.where(dsc, jnp.minimum(a1, b1), jnp.maximum(a1, b1))
                na2 = jnp.where(dsc, jnp.maximum(a2, b2), jnp.minimum(a2, b2))
                nb2 = jnp.where(dsc, jnp.minimum(a2, b2), jnp.maximum(a2, b2))
                v_ref[base:base + j2, :] = jnp.where(
                    dsc, jnp.maximum(na1, na2), jnp.minimum(na1, na2))
                v_ref[base + j2:base + j, :] = jnp.where(
                    dsc, jnp.minimum(na1, na2), jnp.maximum(na1, na2))
                v_ref[base + j:base + j + j2, :] = jnp.where(
                    dsc, jnp.maximum(nb1, nb2), jnp.minimum(nb1, nb2))
                v_ref[base + j + j2:base + 2 * j, :] = jnp.where(
                    dsc, jnp.minimum(nb1, nb2), jnp.maximum(nb1, nb2))
        dsc8 = (jax.lax.broadcasted_iota(jnp.int32, (8, _C), 1) & kc) == 0
        for blk in range(_RA // 16):
            base = blk * 16
            a = v_ref[base:base + 8, :]
            b = v_ref[base + 8:base + 16, :]
            mn = jnp.minimum(a, b)
            mx = jnp.maximum(a, b)
            v_ref[base:base + 8, :] = jnp.where(dsc8, mx, mn)
            v_ref[base + 8:base + 16, :] = jnp.where(dsc8, mn, mx)

        a_small_stages([4, 2, 1], (cA & kc) == 0)
        return carry

    jax.lax.fori_loop(0, 8, a_level_hi, 0)

    rB = jax.lax.broadcasted_iota(jnp.int32, (_C, _C), 0)
    cB = jax.lax.broadcasted_iota(jnp.int32, (_C, _C), 1)

    b = key[_RA:_ROWS, :]
    for m in range(1, 15):
        kB = 1 << m
        for t in range(m - 1, -1, -1):
            j = 1 << t
            if j >= _C:
                js = j // _C
                up = pltpu.roll(b, _C - js, axis=0)
                dn = pltpu.roll(b, js, axis=0)
                lower = (rB & js) == 0
                asc = (rB & (kB // _C)) == 0 if kB >= _C else (cB & kB) == 0
            else:
                up = pltpu.roll(b, _C - j, axis=1)
                dn = pltpu.roll(b, j, axis=1)
                lower = (cB & j) == 0
                asc = (rB & (kB // _C)) == 0 if kB >= _C else (cB & kB) == 0
            part = jnp.where(lower, up, dn)
            mn = jnp.minimum(b, part)
            mx = jnp.maximum(b, part)
            take_max = jnp.logical_xor(lower, asc)
            b = jnp.where(take_max, mx, mn)
    b_sorted = b

    bb = b_sorted.reshape(16, 8, _C)
    parts = [jnp.transpose(bb[:, a, :]) for a in range(8)]
    v_ref[_RA:_R, 112:128] = jnp.concatenate(parts, axis=0)

    c_full = jax.lax.broadcasted_iota(jnp.int32, (_R, _C), 1)
    r_full = jax.lax.broadcasted_iota(jnp.int32, (_R, _C), 0)

    v = v_ref[...]
    lo = v[0:_RA, :]
    hi = v[_RA:_R, :]
    v = jnp.concatenate([jnp.maximum(lo, hi), jnp.minimum(lo, hi)], axis=0)
    for jc in [64, 32, 16, 8, 4, 2, 1]:
        y = pltpu.roll(v, _C - jc, axis=1)
        mn = jnp.minimum(v, y)
        mx = jnp.maximum(v, y)
        send = pltpu.roll(mn, jc, axis=1)
        v = jnp.where((c_full & jc) == 0, mx, send)
    for j in [512, 256, 128, 64, 32, 16, 8]:
        parts = []
        for blk in range(_R // (2 * j)):
            base = blk * 2 * j
            a = v[base:base + j, :]
            b = v[base + j:base + 2 * j, :]
            parts.append(jnp.maximum(a, b))
            parts.append(jnp.minimum(a, b))
        v = jnp.concatenate(parts, axis=0)
    for j in [4, 2, 1]:
        y = pltpu.roll(v, _R - j, axis=0)
        mn = jnp.minimum(v, y)
        mx = jnp.maximum(v, y)
        send = pltpu.roll(mn, j, axis=0)
        v = jnp.where((r_full & j) == 0, mx, send)

    gt = (v & 1).astype(jnp.float32)
    e_sorted = jax.lax.bitcast_convert_type(_sortable(v), jnp.float32)

    cs = gt
    sh = 1
    while sh < _RA:
        blk = jnp.concatenate(
            [jnp.zeros((sh, _C), jnp.float32), cs[:_RA - sh, :],
             jnp.zeros((sh, _C), jnp.float32), cs[_RA:_R - sh, :]], axis=0)
        cs = cs + blk
        sh *= 2
    tot_lo = cs[_RA - 1:_RA, :]
    tot_hi = cs[_R - 1:_R, :]
    run_lo = tot_lo
    run_hi = tot_hi
    sh = 1
    while sh < _C:
        run_lo = run_lo + jnp.concatenate(
            [jnp.zeros((1, sh), jnp.float32), run_lo[:, :-sh]], axis=1)
        run_hi = run_hi + jnp.concatenate(
            [jnp.zeros((1, sh), jnp.float32), run_hi[:, :-sh]], axis=1)
        sh *= 2
    all_lo = run_lo[0:1, _C - 1:_C]
    excl_lo = run_lo - tot_lo
    excl_hi = run_hi - tot_hi + all_lo
    cs = cs + jnp.concatenate(
        [jnp.broadcast_to(excl_lo, (_RA, _C)),
         jnp.broadcast_to(excl_hi, (_RA, _C))], axis=0)

    i_half = (rA_mod := (r_full & (_RA - 1))) + _RA * c_full
    i_glob = i_half + jnp.where(r_full >= _RA, _RA * _C, 0)
    posn = (i_glob + 1).astype(jnp.float32)
    jac = 1.0 - (p - cs) / (p + posn - cs)
    top_lo = jnp.concatenate(
        [jnp.zeros((1, 1), jnp.float32), jac[_RA - 1:_RA, :-1]], axis=1)
    top_hi = jnp.concatenate(
        [jac[_RA - 1:_RA, _C - 1:_C], jac[_R - 1:_R, :-1]], axis=1)
    prevj = jnp.concatenate(
        [top_lo, jac[0:_RA - 1, :], top_hi, jac[_RA:_R - 1, :]], axis=0)
    grad = jac - prevj
    lov = jnp.sum(jnp.maximum(e_sorted, 0.0) * grad)

    contrib = bce_sum / (_N_IMG * _N_PIX) + lov / _N_IMG

    @pl.when(img == 0)
    def _():
        out_ref[...] = jnp.zeros((1, 1), jnp.float32)

    out_ref[...] += jnp.full((1, 1), contrib, jnp.float32)


@jax.jit
def kernel(prediction, target):
    pred = prediction.reshape(_N_IMG, _ROWS, _C)
    tgt = target.reshape(_N_IMG, _ROWS, _C)
    out = pl.pallas_call(
        _lovasz_bce_kernel,
        grid=(_N_IMG,),
        in_specs=[
            pl.BlockSpec((1, _ROWS, _C), lambda i: (i, 0, 0)),
            pl.BlockSpec((1, _ROWS, _C), lambda i: (i, 0, 0)),
        ],
        out_specs=pl.BlockSpec((1, 1), lambda i: (0, 0)),
        out_shape=jax.ShapeDtypeStruct((1, 1), jnp.float32),
        scratch_shapes=[
            pltpu.VMEM((_R, _C), jnp.int32),
        ],
        compiler_params=pltpu.CompilerParams(
            dimension_semantics=("arbitrary",)),
    )(pred, tgt)
    return out[0, 0]
